# baseline (speedup 1.0000x reference)
import functools
import math

import jax
import jax.numpy as jnp
from jax import lax
from jax.experimental import pallas as pl
from jax.experimental.pallas import tpu as pltpu
from jax.experimental.pallas import tpu_sc as plsc

_H = 12


def _mm_body(a_ref, b_ref, bias_ref, o_ref, acc_ref, *, nk, act, bias):
    @pl.when(pl.program_id(2) == 0)
    def _():
        acc_ref[...] = jnp.zeros_like(acc_ref)

    acc_ref[...] += jnp.dot(a_ref[...], b_ref[...],
                            preferred_element_type=jnp.float32)

    @pl.when(pl.program_id(2) == nk - 1)
    def _():
        r = acc_ref[...]
        if bias:
            r = r + bias_ref[...]
        if act == 'relu':
            r = jnp.maximum(r, 0.0)
        elif act == 'tanh':
            r = jnp.tanh(r)
        o_ref[...] = r


def _pad_to(x, m, axis):
    s = x.shape[axis]
    p = (-s) % m
    if p == 0:
        return x
    pads = [(0, 0)] * x.ndim
    pads[axis] = (0, p)
    return jnp.pad(x, pads)


def _mm(a, b, bias=None, act=None):
    M, K = a.shape
    K2, N = b.shape
    assert K == K2
    bm = 256 if M >= 256 else max(8, int(2 ** math.ceil(math.log2(max(M, 1)))))
    bn = 512 if N >= 512 else 128
    bk = 512 if K >= 512 else 128
    ap = _pad_to(_pad_to(a, bm, 0), bk, 1)
    bp = _pad_to(_pad_to(b, bk, 0), bn, 1)
    Mp, Kp = ap.shape
    Np = bp.shape[1]
    has_bias = bias is not None
    if has_bias:
        biasp = _pad_to(bias.reshape(1, -1), bn, 1)
    else:
        biasp = jnp.zeros((1, bn), jnp.float32)
    grid = (Mp // bm, Np // bn, Kp // bk)
    out = pl.pallas_call(
        functools.partial(_mm_body, nk=grid[2], act=act, bias=has_bias),
        grid=grid,
        in_specs=[
            pl.BlockSpec((bm, bk), lambda i, j, k: (i, k)),
            pl.BlockSpec((bk, bn), lambda i, j, k: (k, j)),
            pl.BlockSpec((1, bn), lambda i, j, k: (0, j)),
        ],
        out_specs=pl.BlockSpec((bm, bn), lambda i, j, k: (i, j)),
        out_shape=jax.ShapeDtypeStruct((Mp, Np), jnp.float32),
        scratch_shapes=[pltpu.VMEM((bm, bn), jnp.float32)],
        compiler_params=pltpu.CompilerParams(
            dimension_semantics=('parallel', 'parallel', 'arbitrary')),
    )(ap, bp, biasp)
    return out[:M, :N]


_B1 = 8
_EPAD = 4096


@functools.cache
def _sc_edge_messages(E2p, H, C):
    TW = H * C
    nb = E2p // 32 // _B1
    mesh = plsc.VectorSubcoreMesh(core_axis_name="c", subcore_axis_name="s")

    @functools.partial(
        pl.kernel, mesh=mesh,
        out_type=jax.ShapeDtypeStruct((E2p, C), jnp.float32),
        scratch_types=[
            pltpu.VMEM((_B1,), jnp.int32),
            pltpu.VMEM((_B1, TW), jnp.float32),
            pltpu.VMEM((_B1, 16), jnp.float32),
            pltpu.VMEM((_B1, C), jnp.float32),
            pltpu.SemaphoreType.DMA,
        ],
    )
    def k(table_hbm, row_hbm, coef_hbm, msg_hbm, idx_v, rows_v, coef_v,
          msg_v, sem):
        wid = lax.axis_index("s") * 2 + lax.axis_index("c")

        def body(bi, carry):
            e0 = (wid * nb + bi) * _B1
            pltpu.sync_copy(row_hbm.at[pl.ds(e0, _B1)], idx_v)
            pltpu.async_copy(table_hbm.at[idx_v], rows_v, sem).wait()
            pltpu.sync_copy(coef_hbm.at[pl.ds(e0, _B1)], coef_v)
            for b in range(_B1):
                cvec = coef_v[b, pl.ds(0, 16)]
                sc = [cvec[h] for h in range(H)]

                def jbody(j, c2):
                    acc = sc[0] * rows_v[b, pl.ds(j * 16, 16)]
                    for h in range(1, H):
                        acc = acc + sc[h] * rows_v[b, pl.ds(h * C + j * 16, 16)]
                    msg_v[b, pl.ds(j * 16, 16)] = acc
                    return c2

                lax.fori_loop(0, C // 16, jbody, 0)
            pltpu.sync_copy(msg_v, msg_hbm.at[pl.ds(e0, _B1)])
            return carry

        lax.fori_loop(0, nb, body, 0)

    return k


_SSEG = 2048


@functools.cache
def _sc_scatter_add(E2p, C, R, P):
    nseg = E2p // _SSEG
    mesh = plsc.VectorSubcoreMesh(core_axis_name="c", subcore_axis_name="s")

    @functools.partial(
        pl.kernel, mesh=mesh,
        out_type=jax.ShapeDtypeStruct((32 * P * R, C), jnp.float32),
        scratch_types=[
            pltpu.VMEM((_SSEG,), jnp.int32),
            pltpu.VMEM((1, C), jnp.float32),
            pltpu.VMEM((R, C), jnp.float32),
            pltpu.SemaphoreType.DMA,
        ],
    )
    def k(msg_hbm, col_hbm, out_hbm, colb, rowb, acc, sem):
        wid = lax.axis_index("s") * 2 + lax.axis_index("c")
        for p in range(P):
            g0 = (p * 32 + wid) * R

            def zb(i, carry):
                r = i // (C // 16)
                jj = i % (C // 16)
                acc[r, pl.ds(jj * 16, 16)] = jnp.zeros((16,), jnp.float32)
                return carry

            lax.fori_loop(0, R * (C // 16), zb, 0)

            def seg(si, carry):
                e0 = si * _SSEG
                pltpu.sync_copy(col_hbm.at[pl.ds(e0, _SSEG)], colb)

                def scan(vi, c2):
                    cv = colb[pl.ds(vi * 16, 16)]
                    lv = cv - g0
                    for b in range(16):
                        lvb = lv[b]

                        @pl.when((lvb >= 0) & (lvb < R))
                        def _():
                            eid = e0 + vi * 16 + b
                            pltpu.sync_copy(msg_hbm.at[pl.ds(eid, 1)], rowb)

                            def rmw(jj, c3):
                                sl = pl.ds(jj * 16, 16)
                                acc[lvb, sl] = acc[lvb, sl] + rowb[0, sl]
                                return c3

                            lax.fori_loop(0, C // 16, rmw, 0)
                    return c2

                lax.fori_loop(0, _SSEG // 16, scan, 0)
                return carry

            lax.fori_loop(0, nseg, seg, 0)
            pltpu.sync_copy(acc, out_hbm.at[pl.ds(g0, R)])

    return k


def _range_plan(N, C):
    if C == 512:
        return 160, 2
    return 80, 1


def _segment_sum_sc(msg, col, N, C):
    E2p = msg.shape[0]
    R, P = _range_plan(N, C)
    out = _sc_scatter_add(E2p, C, R, P)(msg, col)
    return out[:N]


def _pad_edges(row, col, coef):
    E = row.shape[0]
    Ep = 172032
    assert E <= Ep
    rp = jnp.pad(row, (0, Ep - E))
    cp = jnp.pad(col, (0, Ep - E), constant_values=-1)
    H = coef.shape[1]
    kp = jnp.pad(coef, ((0, Ep - E), (0, 16 - H)))
    return rp, cp, kp


def _gat(x, row, col, valid, W, att_src, att_dst, C):
    N = x.shape[0]
    Cin = x.shape[1]
    h = _mm(x, W)
    W3 = W.reshape(Cin, _H, C)
    wsd = jnp.concatenate([jnp.einsum('ihc,hc->ih', W3, att_src),
                           jnp.einsum('ihc,hc->ih', W3, att_dst)], axis=1)
    aa = _mm(x, wsd)
    a_src, a_dst = aa[:, :_H], aa[:, _H:]

    loop = jnp.arange(N, dtype=row.dtype)
    row2 = jnp.concatenate([row, loop])
    col2 = jnp.concatenate([col, loop])
    valid2 = jnp.concatenate([valid, jnp.ones((N,), dtype=bool)])

    alpha = jax.nn.leaky_relu(a_src[row2] + a_dst[col2], negative_slope=0.2)
    alpha = jnp.where(valid2[:, None], alpha, -1e9)
    amax = jax.ops.segment_max(alpha, col2, num_segments=N)
    ex = jnp.exp(alpha - amax[col2]) * valid2[:, None].astype(alpha.dtype)
    denom = jax.ops.segment_sum(ex, col2, num_segments=N)
    coef = ex / (denom[col2] + 1e-16) * (1.0 / _H)

    rp, cp, kp = _pad_edges(row2, col2, coef)
    msg = _sc_edge_messages(rp.shape[0], _H, C)(h, rp, kp)
    return _segment_sum_sc(msg, cp, N, C)


def _sag_pool(x, row, col, valid, batch, Wrel, brel, Wroot):
    N, C = x.shape
    vf = valid[:, None].astype(x.dtype)
    rp, cp, kp = _pad_edges(row, col, vf)
    aggmsg = _sc_edge_messages(rp.shape[0], 1, C)(x, rp, kp)
    agg = _segment_sum_sc(aggmsg, cp, N, C)
    sc_in = jnp.concatenate([agg, x], axis=1)
    sc_w = jnp.concatenate([Wrel, Wroot], axis=0)
    score = _mm(sc_in, sc_w, bias=brel, act='tanh')[:, 0]
    k = int(math.ceil(0.5 * N))
    sv, perm = jax.lax.top_k(score, k)
    x_new = x[perm] * sv[:, None]
    batch_new = batch[perm]
    new_idx = jnp.full((N,), -1, dtype=jnp.int32).at[perm].set(
        jnp.arange(k, dtype=jnp.int32))
    row_n = new_idx[row]
    col_n = new_idx[col]
    valid_n = valid & (row_n >= 0) & (col_n >= 0)
    row_n = jnp.where(valid_n, row_n, 0)
    col_n = jnp.where(valid_n, col_n, 0)
    return x_new, row_n, col_n, valid_n, batch_new


def kernel(esm_rep, seq, pssm, A, seq_embed, batch, params):
    p = params
    embed = _mm(seq[0].T, p['W_seq'].T, bias=p['b_seq'], act='relu')
    row = A[0].astype(jnp.int32)
    col = A[1].astype(jnp.int32)
    valid = jnp.ones((row.shape[0],), dtype=bool)
    b = batch.astype(jnp.int32)

    out = _gat(embed, row, col, valid, p['Wg1'], p['as1'], p['ad1'], 512)
    out, row, col, valid, b = _sag_pool(out, row, col, valid, b,
                                        p['Wrel1'], p['brel1'], p['Wroot1'])
    out = _gat(out, row, col, valid, p['Wg2'], p['as2'], p['ad2'], 512)
    out, row, col, valid, b = _sag_pool(out, row, col, valid, b,
                                        p['Wrel2'], p['brel2'], p['Wroot2'])
    out = _gat(out, row, col, valid, p['Wg3'], p['as3'], p['ad3'], 1024)
    out, row, col, valid, b = _sag_pool(out, row, col, valid, b,
                                        p['Wrel3'], p['brel3'], p['Wroot3'])
    out = _gat(out, row, col, valid, p['Wg4'], p['as4'], p['ad4'], 1024)
    out, row, col, valid, b = _sag_pool(out, row, col, valid, b,
                                        p['Wrel4'], p['brel4'], p['Wroot4'])

    pooled = jnp.mean(out, axis=0, keepdims=True)
    feat = jnp.concatenate([pooled, seq_embed], axis=1)
    hdn = _mm(feat, p['Wc1'], bias=p['bc1'], act='relu')
    return _mm(hdn, p['Wc2'], bias=p['bc2'])


# speedup vs baseline: 1.3919x; 1.3919x over previous
import functools
import math

import jax
import jax.numpy as jnp
from jax import lax
from jax.experimental import pallas as pl
from jax.experimental.pallas import tpu as pltpu
from jax.experimental.pallas import tpu_sc as plsc

_H = 12


def _mm_body(a_ref, b_ref, bias_ref, o_ref, acc_ref, *, nk, act, bias):
    @pl.when(pl.program_id(2) == 0)
    def _():
        acc_ref[...] = jnp.zeros_like(acc_ref)

    acc_ref[...] += jnp.dot(a_ref[...], b_ref[...],
                            preferred_element_type=jnp.float32)

    @pl.when(pl.program_id(2) == nk - 1)
    def _():
        r = acc_ref[...]
        if bias:
            r = r + bias_ref[...]
        if act == 'relu':
            r = jnp.maximum(r, 0.0)
        elif act == 'tanh':
            r = jnp.tanh(r)
        o_ref[...] = r


def _pad_to(x, m, axis):
    s = x.shape[axis]
    p = (-s) % m
    if p == 0:
        return x
    pads = [(0, 0)] * x.ndim
    pads[axis] = (0, p)
    return jnp.pad(x, pads)


def _mm(a, b, bias=None, act=None):
    M, K = a.shape
    K2, N = b.shape
    assert K == K2
    bm = 256 if M >= 256 else max(8, int(2 ** math.ceil(math.log2(max(M, 1)))))
    bn = 512 if N >= 512 else 128
    bk = 512 if K >= 512 else 128
    ap = _pad_to(_pad_to(a, bm, 0), bk, 1)
    bp = _pad_to(_pad_to(b, bk, 0), bn, 1)
    Mp, Kp = ap.shape
    Np = bp.shape[1]
    has_bias = bias is not None
    if has_bias:
        biasp = _pad_to(bias.reshape(1, -1), bn, 1)
    else:
        biasp = jnp.zeros((1, bn), jnp.float32)
    grid = (Mp // bm, Np // bn, Kp // bk)
    out = pl.pallas_call(
        functools.partial(_mm_body, nk=grid[2], act=act, bias=has_bias),
        grid=grid,
        in_specs=[
            pl.BlockSpec((bm, bk), lambda i, j, k: (i, k)),
            pl.BlockSpec((bk, bn), lambda i, j, k: (k, j)),
            pl.BlockSpec((1, bn), lambda i, j, k: (0, j)),
        ],
        out_specs=pl.BlockSpec((bm, bn), lambda i, j, k: (i, j)),
        out_shape=jax.ShapeDtypeStruct((Mp, Np), jnp.float32),
        scratch_shapes=[pltpu.VMEM((bm, bn), jnp.float32)],
        compiler_params=pltpu.CompilerParams(
            dimension_semantics=('parallel', 'parallel', 'arbitrary')),
    )(ap, bp, biasp)
    return out[:M, :N]


@functools.cache
def _sc_edge_messages(E2p, H, C):
    TW = H * C
    B1 = min(64, max(8, (430080 // ((TW + C) * 4)) // 8 * 8))
    nb = E2p // 32 // B1
    mesh = plsc.VectorSubcoreMesh(core_axis_name="c", subcore_axis_name="s")

    @functools.partial(
        pl.kernel, mesh=mesh,
        out_type=jax.ShapeDtypeStruct((E2p, C), jnp.float32),
        scratch_types=[
            pltpu.VMEM((B1,), jnp.int32),
            pltpu.VMEM((B1, TW), jnp.float32),
            pltpu.VMEM((B1, 16), jnp.float32),
            pltpu.VMEM((B1, C), jnp.float32),
            pltpu.SemaphoreType.DMA,
        ],
    )
    def k(table_hbm, row_hbm, coef_hbm, msg_hbm, idx_v, rows_v, coef_v,
          msg_v, sem):
        wid = lax.axis_index("s") * 2 + lax.axis_index("c")

        def body(bi, carry):
            e0 = (wid * nb + bi) * B1
            pltpu.sync_copy(row_hbm.at[pl.ds(e0, B1)], idx_v)
            pltpu.async_copy(table_hbm.at[idx_v], rows_v, sem).wait()
            pltpu.sync_copy(coef_hbm.at[pl.ds(e0, B1)], coef_v)
            for b in range(B1):
                cvec = coef_v[b, pl.ds(0, 16)]
                sc = [cvec[h] for h in range(H)]

                def jbody(j, c2):
                    acc = sc[0] * rows_v[b, pl.ds(j * 16, 16)]
                    for h in range(1, H):
                        acc = acc + sc[h] * rows_v[b, pl.ds(h * C + j * 16, 16)]
                    msg_v[b, pl.ds(j * 16, 16)] = acc
                    return c2

                lax.fori_loop(0, C // 16, jbody, 0)
            pltpu.sync_copy(msg_v, msg_hbm.at[pl.ds(e0, B1)])
            return carry

        lax.fori_loop(0, nb, body, 0)

    return k


_SSEG = 2048


@functools.cache
def _sc_scatter_add(E2p, C, R, P):
    nseg = E2p // _SSEG
    mesh = plsc.VectorSubcoreMesh(core_axis_name="c", subcore_axis_name="s")

    @functools.partial(
        pl.kernel, mesh=mesh,
        out_type=jax.ShapeDtypeStruct((32 * P * R, C), jnp.float32),
        scratch_types=[
            pltpu.VMEM((_SSEG,), jnp.int32),
            pltpu.VMEM((16, C), jnp.float32),
            pltpu.VMEM((R, C), jnp.float32),
            pltpu.SemaphoreType.DMA,
        ],
    )
    def k(msg_hbm, col_hbm, out_hbm, colb, rows16, acc, sem):
        wid = lax.axis_index("s") * 2 + lax.axis_index("c")
        lane = lax.iota(jnp.int32, 16)
        for p in range(P):
            g0 = (p * 32 + wid) * R

            def zb(i, carry):
                r = i // (C // 16)
                jj = i % (C // 16)
                acc[r, pl.ds(jj * 16, 16)] = jnp.zeros((16,), jnp.float32)
                return carry

            lax.fori_loop(0, R * (C // 16), zb, 0)

            def seg(si, carry):
                e0 = si * _SSEG
                pltpu.sync_copy(col_hbm.at[pl.ds(e0, _SSEG)], colb)

                def scan(vi, c2):
                    cv = colb[pl.ds(vi * 16, 16)]
                    lv = cv - g0
                    m = (lv >= 0) & (lv < R)
                    lvs = [lv[b] for b in range(16)]
                    ms = [(x >= 0) & (x < R) for x in lvs]
                    hit = ms[0]
                    for b in range(1, 16):
                        hit = hit | ms[b]

                    @pl.when(hit)
                    def _():
                        idxv = jnp.where(m, e0 + vi * 16 + lane, 0)
                        pltpu.async_copy(msg_hbm.at[idxv], rows16, sem).wait()
                        for b in range(16):
                            lvb = lvs[b]

                            @pl.when(ms[b])
                            def _():
                                def rmw(jj, c3):
                                    sl = pl.ds(jj * 16, 16)
                                    acc[lvb, sl] = acc[lvb, sl] + rows16[b, sl]
                                    return c3

                                lax.fori_loop(0, C // 16, rmw, 0)
                    return c2

                lax.fori_loop(0, _SSEG // 16, scan, 0)
                return carry

            lax.fori_loop(0, nseg, seg, 0)
            pltpu.sync_copy(acc, out_hbm.at[pl.ds(g0, R)])

    return k


def _range_plan(N, C):
    if C == 512:
        return 160, 2
    return 80, 1


def _segment_sum_sc(msg, col, N, C):
    E2p = msg.shape[0]
    R, P = _range_plan(N, C)
    out = _sc_scatter_add(E2p, C, R, P)(msg, col)
    return out[:N]


def _pad_edges(row, col, coef):
    E = row.shape[0]
    Ep = 172032
    assert E <= Ep
    rp = jnp.pad(row, (0, Ep - E))
    cp = jnp.pad(col, (0, Ep - E), constant_values=-1)
    H = coef.shape[1]
    kp = jnp.pad(coef, ((0, Ep - E), (0, 16 - H)))
    return rp, cp, kp


def _gat(x, row, col, valid, W, att_src, att_dst, C):
    N = x.shape[0]
    Cin = x.shape[1]
    h = _mm(x, W)
    W3 = W.reshape(Cin, _H, C)
    wsd = jnp.concatenate([jnp.einsum('ihc,hc->ih', W3, att_src),
                           jnp.einsum('ihc,hc->ih', W3, att_dst)], axis=1)
    aa = _mm(x, wsd)
    a_src, a_dst = aa[:, :_H], aa[:, _H:]

    loop = jnp.arange(N, dtype=row.dtype)
    row2 = jnp.concatenate([row, loop])
    col2 = jnp.concatenate([col, loop])
    valid2 = jnp.concatenate([valid, jnp.ones((N,), dtype=bool)])

    alpha = jax.nn.leaky_relu(a_src[row2] + a_dst[col2], negative_slope=0.2)
    alpha = jnp.where(valid2[:, None], alpha, -1e9)
    amax = jax.ops.segment_max(alpha, col2, num_segments=N)
    ex = jnp.exp(alpha - amax[col2]) * valid2[:, None].astype(alpha.dtype)
    denom = jax.ops.segment_sum(ex, col2, num_segments=N)
    coef = ex / (denom[col2] + 1e-16) * (1.0 / _H)

    rp, cp, kp = _pad_edges(row2, col2, coef)
    msg = _sc_edge_messages(rp.shape[0], _H, C)(h, rp, kp)
    return _segment_sum_sc(msg, cp, N, C)


def _sag_pool(x, row, col, valid, batch, Wrel, brel, Wroot):
    N, C = x.shape
    vf = valid[:, None].astype(x.dtype)
    rp, cp, kp = _pad_edges(row, col, vf)
    aggmsg = _sc_edge_messages(rp.shape[0], 1, C)(x, rp, kp)
    agg = _segment_sum_sc(aggmsg, cp, N, C)
    sc_in = jnp.concatenate([agg, x], axis=1)
    sc_w = jnp.concatenate([Wrel, Wroot], axis=0)
    score = _mm(sc_in, sc_w, bias=brel, act='tanh')[:, 0]
    k = int(math.ceil(0.5 * N))
    sv, perm = jax.lax.top_k(score, k)
    x_new = x[perm] * sv[:, None]
    batch_new = batch[perm]
    new_idx = jnp.full((N,), -1, dtype=jnp.int32).at[perm].set(
        jnp.arange(k, dtype=jnp.int32))
    row_n = new_idx[row]
    col_n = new_idx[col]
    valid_n = valid & (row_n >= 0) & (col_n >= 0)
    row_n = jnp.where(valid_n, row_n, 0)
    col_n = jnp.where(valid_n, col_n, 0)
    return x_new, row_n, col_n, valid_n, batch_new


def kernel(esm_rep, seq, pssm, A, seq_embed, batch, params):
    p = params
    embed = _mm(seq[0].T, p['W_seq'].T, bias=p['b_seq'], act='relu')
    row = A[0].astype(jnp.int32)
    col = A[1].astype(jnp.int32)
    valid = jnp.ones((row.shape[0],), dtype=bool)
    b = batch.astype(jnp.int32)

    out = _gat(embed, row, col, valid, p['Wg1'], p['as1'], p['ad1'], 512)
    out, row, col, valid, b = _sag_pool(out, row, col, valid, b,
                                        p['Wrel1'], p['brel1'], p['Wroot1'])
    out = _gat(out, row, col, valid, p['Wg2'], p['as2'], p['ad2'], 512)
    out, row, col, valid, b = _sag_pool(out, row, col, valid, b,
                                        p['Wrel2'], p['brel2'], p['Wroot2'])
    out = _gat(out, row, col, valid, p['Wg3'], p['as3'], p['ad3'], 1024)
    out, row, col, valid, b = _sag_pool(out, row, col, valid, b,
                                        p['Wrel3'], p['brel3'], p['Wroot3'])
    out = _gat(out, row, col, valid, p['Wg4'], p['as4'], p['ad4'], 1024)
    out, row, col, valid, b = _sag_pool(out, row, col, valid, b,
                                        p['Wrel4'], p['brel4'], p['Wroot4'])

    pooled = jnp.mean(out, axis=0, keepdims=True)
    feat = jnp.concatenate([pooled, seq_embed], axis=1)
    hdn = _mm(feat, p['Wc1'], bias=p['bc1'], act='relu')
    return _mm(hdn, p['Wc2'], bias=p['bc2'])
